# baseline (speedup 1.0000x reference)
import functools

import jax
import jax.numpy as jnp
from jax import lax
from jax.experimental import pallas as pl
from jax.experimental.pallas import tpu as pltpu
from jax.experimental.pallas import tpu_sc as plsc
from jax.experimental.layout import Format, Layout

_NUM_CORES = 2
_NUM_SUBCORES = 16
_NUM_WORKERS = _NUM_CORES * _NUM_SUBCORES
_CHUNK = 128


def _gather_sc(idx3, table):
    nw, n_chunks, chunk = idx3.shape
    b_per_w = n_chunks * chunk
    _, d = table.shape

    mesh = plsc.VectorSubcoreMesh(
        core_axis_name="c",
        subcore_axis_name="s",
        num_cores=_NUM_CORES,
        num_subcores=_NUM_SUBCORES,
    )

    @functools.partial(
        pl.kernel,
        out_type=jax.ShapeDtypeStruct((nw * b_per_w, d), jnp.float32),
        mesh=mesh,
        scratch_types=[
            pltpu.VMEM((n_chunks, chunk), jnp.int32),
            pltpu.VMEM((b_per_w, d), jnp.float32),
            pltpu.SemaphoreType.DMA,
        ],
        compiler_params=pltpu.CompilerParams(use_tc_tiling_on_sc=False),
    )
    def k(idx_hbm, table_hbm, out_hbm, idx_v, rows_v, sem):
        wid = lax.axis_index("s") * _NUM_CORES + lax.axis_index("c")
        pltpu.sync_copy(idx_hbm.at[wid], idx_v)
        copies = []
        for j in range(n_chunks):
            copies.append(
                pltpu.async_copy(
                    table_hbm.at[idx_v.at[j]],
                    rows_v.at[pl.ds(j * chunk, chunk)],
                    sem,
                )
            )
        for c in copies:
            c.wait()
        pltpu.sync_copy(rows_v, out_hbm.at[pl.ds(wid * b_per_w, b_per_w)])

    return k(idx3, table)


def kernel(inputs, length, mode, table):
    b = inputs.shape[0]
    assert b % (_NUM_WORKERS * _CHUNK) == 0, b
    n_chunks = b // (_NUM_WORKERS * _CHUNK)
    idx3 = inputs.reshape(_NUM_WORKERS, n_chunks, _CHUNK)
    table_lin = jax.device_put(
        table,
        Format(
            Layout(major_to_minor=(0, 1), tiling=((8,),)),
            jax.sharding.SingleDeviceSharding(jax.devices()[0]),
        ),
    )
    return _gather_sc(idx3, table_lin)
